# baseline (speedup 1.0000x reference)
import functools

import jax
import jax.numpy as jnp
from jax import lax
from jax.experimental import pallas as pl
from jax.experimental.pallas import tpu as pltpu
from jax.experimental.pallas import tpu_sc as plsc

N = 10000
E = 320000
D = 128
C_OUT = 47
C_PAD = 48

NC = 2
NS = 16
NW = NC * NS
CH = 128
T = 80
E_PAD = NW * T * CH
N_PAD = 10240
ROWS_PER_TILE = N_PAD // NS
CP_PER_TILE = ROWS_PER_TILE // CH

_MESH = plsc.VectorSubcoreMesh(
    core_axis_name="c", subcore_axis_name="s", num_cores=NC, num_subcores=NS
)


def _zero_rows(rows):
    def zrow(i, carry):
        for j in range(D // 16):
            rows[i, pl.ds(j * 16, 16)] = jnp.zeros((16,), jnp.float32)
        return carry

    lax.fori_loop(0, CH, zrow, 0)


def _agg_body(tab_hbm, src_hbm, dst_hbm, out_hbm,
              idx_s, idx_d, rows0, rows1, acc, sg0, sg1, ss0, ss1):
    c = lax.axis_index("c")
    s = lax.axis_index("s")
    wid = c * NS + s

    _zero_rows(rows0)
    for k in range(CP_PER_TILE):
        pltpu.sync_copy(rows0, acc.at[pl.ds(s * ROWS_PER_TILE + k * CH, CH)])
    plsc.subcore_barrier()

    base = wid * (T * CH)

    def edge_chunk(i, carry):
        off = base + i * CH
        pltpu.sync_copy(src_hbm.at[pl.ds(off, CH)], idx_s)
        pltpu.sync_copy(dst_hbm.at[pl.ds(off, CH)], idx_d)
        pltpu.async_copy(tab_hbm.at[idx_s], rows0, sg0).wait()
        pltpu.sync_copy(rows0, acc.at[idx_d], add=True)
        return carry

    lax.fori_loop(0, T, edge_chunk, 0)
    plsc.subcore_barrier()

    for k in range(CP_PER_TILE):
        r0 = s * ROWS_PER_TILE + k * CH
        pltpu.sync_copy(acc.at[pl.ds(r0, CH)], rows0)
        pltpu.sync_copy(rows0, out_hbm.at[pl.ds(c * N_PAD + r0, CH)])


def _deg_body(dst_hbm, out_hbm, idx_d, rows, acc, sem):
    c = lax.axis_index("c")
    s = lax.axis_index("s")
    wid = c * NS + s

    _zero_rows(rows)
    for k in range(CP_PER_TILE):
        pltpu.sync_copy(rows, acc.at[pl.ds(s * ROWS_PER_TILE + k * CH, CH)])
    plsc.subcore_barrier()

    def fill1(i, carry):
        for j in range(D // 16):
            rows[i, pl.ds(j * 16, 16)] = jnp.ones((16,), jnp.float32)
        return carry

    lax.fori_loop(0, CH, fill1, 0)

    base = wid * (T * CH)

    def edge_chunk(i, carry):
        off = base + i * CH
        pltpu.sync_copy(dst_hbm.at[pl.ds(off, CH)], idx_d)
        pltpu.sync_copy(rows, acc.at[idx_d], add=True)
        return carry

    lax.fori_loop(0, T, edge_chunk, 0)
    plsc.subcore_barrier()

    for k in range(CP_PER_TILE):
        r0 = s * ROWS_PER_TILE + k * CH
        pltpu.sync_copy(acc.at[pl.ds(r0, CH)], rows)
        pltpu.sync_copy(rows, out_hbm.at[pl.ds(c * N_PAD + r0, CH)])


_sc_agg = pl.kernel(
    _agg_body,
    out_type=[jax.ShapeDtypeStruct((NC * N_PAD, D), jnp.float32)],
    mesh=_MESH,
    scratch_types=(
        [pltpu.VMEM((CH,), jnp.int32)] * 2
        + [pltpu.VMEM((CH, D), jnp.float32)] * 2
        + [pltpu.VMEM_SHARED((N_PAD, D), jnp.float32)]
        + [pltpu.SemaphoreType.DMA] * 4
    ),
)

_sc_deg = pl.kernel(
    _deg_body,
    out_type=[jax.ShapeDtypeStruct((NC * N_PAD, D), jnp.float32)],
    mesh=_MESH,
    scratch_types=[
        pltpu.VMEM((CH,), jnp.int32),
        pltpu.VMEM((CH, D), jnp.float32),
        pltpu.VMEM_SHARED((N_PAD, D), jnp.float32),
        pltpu.SemaphoreType.DMA,
    ],
)

R = 1000


def _combine_body(relu, p_ref, d_ref, x_ref, ws_ref, wn_ref, b_ref, o_ref):
    deg = jnp.maximum(d_ref[0, :, 0] + d_ref[1, :, 0], 1.0)
    neigh = (p_ref[0] + p_ref[1]) / deg[:, None]
    h = (
        jnp.dot(x_ref[...], ws_ref[...], preferred_element_type=jnp.float32)
        + jnp.dot(neigh, wn_ref[...], preferred_element_type=jnp.float32)
        + b_ref[...]
    )
    if relu:
        h = jnp.maximum(h, 0.0)
    o_ref[...] = h


def _make_tc_combine(relu: bool, d_out: int):
    return pl.pallas_call(
        functools.partial(_combine_body, relu),
        grid=(N // R,),
        in_specs=[
            pl.BlockSpec((NC, R, D), lambda i: (0, i, 0)),
            pl.BlockSpec((NC, R, 1), lambda i: (0, i, 0)),
            pl.BlockSpec((R, D), lambda i: (i, 0)),
            pl.BlockSpec((D, d_out), lambda i: (0, 0)),
            pl.BlockSpec((D, d_out), lambda i: (0, 0)),
            pl.BlockSpec((1, d_out), lambda i: (0, 0)),
        ],
        out_shape=jax.ShapeDtypeStruct((N, d_out), jnp.float32),
        out_specs=pl.BlockSpec((R, d_out), lambda i: (i, 0)),
    )


_tc_combine_relu = _make_tc_combine(True, D)
_tc_final = _make_tc_combine(False, C_PAD)


def kernel(x, edge_index, W_self_0, W_neigh_0, b_0, W_self_1, W_neigh_1, b_1,
           W_self_2, W_neigh_2, b_2):
    pad = E_PAD - E
    src = jnp.concatenate([edge_index[0], jnp.zeros((pad,), jnp.int32)])
    dst = jnp.concatenate(
        [edge_index[1], N + (jnp.arange(pad, dtype=jnp.int32) % (N_PAD - N))]
    )

    ws2 = jnp.pad(W_self_2, ((0, 0), (0, C_PAD - C_OUT)))
    wn2 = jnp.pad(W_neigh_2, ((0, 0), (0, C_PAD - C_OUT)))
    b2 = jnp.pad(b_2, (0, C_PAD - C_OUT)).reshape(1, C_PAD)

    (degf,) = _sc_deg(dst)
    degp = degf.reshape(NC, N_PAD, D)[:, :, 0:1]

    (p0,) = _sc_agg(x, src, dst)
    p0 = p0.reshape(NC, N_PAD, D)
    h1 = _tc_combine_relu(p0, degp, x, W_self_0, W_neigh_0, b_0.reshape(1, D))

    (p1,) = _sc_agg(h1, src, dst)
    p1 = p1.reshape(NC, N_PAD, D)
    h2 = _tc_combine_relu(p1, degp, h1, W_self_1, W_neigh_1, b_1.reshape(1, D))

    (p2,) = _sc_agg(h2, src, dst)
    p2 = p2.reshape(NC, N_PAD, D)
    out = _tc_final(p2, degp, h2, ws2, wn2, b2)
    return out[:, :C_OUT]


# speedup vs baseline: 2.0902x; 2.0902x over previous
import functools

import jax
import jax.numpy as jnp
from jax import lax
from jax.experimental import pallas as pl
from jax.experimental.pallas import tpu as pltpu
from jax.experimental.pallas import tpu_sc as plsc

N = 10000
E = 320000
D = 128
C_OUT = 47
C_PAD = 48

NC = 2
NS = 16
NW = NC * NS
CH = 128
T = 80
E_PAD = NW * T * CH
N_PAD = 10240
ROWS_PER_TILE = N_PAD // NS
CP_PER_TILE = ROWS_PER_TILE // CH

_MESH = plsc.VectorSubcoreMesh(
    core_axis_name="c", subcore_axis_name="s", num_cores=NC, num_subcores=NS
)


def _zero_rows(rows):
    def zrow(i, carry):
        for j in range(D // 16):
            rows[i, pl.ds(j * 16, 16)] = jnp.zeros((16,), jnp.float32)
        return carry

    lax.fori_loop(0, CH, zrow, 0)


def _agg_body(tab_hbm, src_hbm, dst_hbm, out_hbm,
              idx_s, idx_d, rows0, rows1, acc, sg0, sg1, ss0, ss1):
    c = lax.axis_index("c")
    s = lax.axis_index("s")
    wid = c * NS + s

    _zero_rows(rows0)
    for k in range(CP_PER_TILE):
        pltpu.sync_copy(rows0, acc.at[pl.ds(s * ROWS_PER_TILE + k * CH, CH)])
    plsc.subcore_barrier()

    base = wid * (T * CH)

    def edge_chunk(i, carry):
        off = base + i * CH
        pltpu.sync_copy(src_hbm.at[pl.ds(off, CH)], idx_s)
        pltpu.sync_copy(dst_hbm.at[pl.ds(off, CH)], idx_d)
        pltpu.async_copy(tab_hbm.at[idx_s], rows0, sg0).wait()
        pltpu.sync_copy(rows0, acc.at[idx_d], add=True)
        return carry

    lax.fori_loop(0, T, edge_chunk, 0)
    plsc.subcore_barrier()

    for k in range(CP_PER_TILE):
        r0 = s * ROWS_PER_TILE + k * CH
        pltpu.sync_copy(acc.at[pl.ds(r0, CH)], rows0)
        pltpu.sync_copy(rows0, out_hbm.at[pl.ds(c * N_PAD + r0, CH)])


def _deg_body(dst_hbm, out_hbm, idx_d, rows, acc, sem):
    c = lax.axis_index("c")
    s = lax.axis_index("s")
    wid = c * NS + s

    _zero_rows(rows)
    for k in range(CP_PER_TILE):
        pltpu.sync_copy(rows, acc.at[pl.ds(s * ROWS_PER_TILE + k * CH, CH)])
    plsc.subcore_barrier()

    def fill1(i, carry):
        for j in range(D // 16):
            rows[i, pl.ds(j * 16, 16)] = jnp.ones((16,), jnp.float32)
        return carry

    lax.fori_loop(0, CH, fill1, 0)

    base = wid * (T * CH)

    def edge_chunk(i, carry):
        off = base + i * CH
        pltpu.sync_copy(dst_hbm.at[pl.ds(off, CH)], idx_d)
        pltpu.sync_copy(rows, acc.at[idx_d], add=True)
        return carry

    lax.fori_loop(0, T, edge_chunk, 0)
    plsc.subcore_barrier()

    for k in range(CP_PER_TILE):
        r0 = s * ROWS_PER_TILE + k * CH
        pltpu.sync_copy(acc.at[pl.ds(r0, CH)], rows)
        pltpu.sync_copy(rows, out_hbm.at[pl.ds(c * N_PAD + r0, CH)])


_sc_agg = pl.kernel(
    _agg_body,
    out_type=[jax.ShapeDtypeStruct((NC * N_PAD, D), jnp.float32)],
    mesh=_MESH,
    scratch_types=(
        [pltpu.VMEM((CH,), jnp.int32)] * 2
        + [pltpu.VMEM((CH, D), jnp.float32)] * 2
        + [pltpu.VMEM_SHARED((N_PAD, D), jnp.float32)]
        + [pltpu.SemaphoreType.DMA] * 4
    ),
)

_sc_deg = pl.kernel(
    _deg_body,
    out_type=[jax.ShapeDtypeStruct((NC * N_PAD, D), jnp.float32)],
    mesh=_MESH,
    scratch_types=[
        pltpu.VMEM((CH,), jnp.int32),
        pltpu.VMEM((CH, D), jnp.float32),
        pltpu.VMEM_SHARED((N_PAD, D), jnp.float32),
        pltpu.SemaphoreType.DMA,
    ],
)

R = 1000


def _combine_body(relu, p_ref, d_ref, x_ref, ws_ref, wn_ref, b_ref, o_ref):
    deg = jnp.maximum(d_ref[0, :, 0] + d_ref[1, :, 0], 1.0)
    neigh = (p_ref[0] + p_ref[1]) / deg[:, None]
    h = (
        jnp.dot(x_ref[...], ws_ref[...], preferred_element_type=jnp.float32)
        + jnp.dot(neigh, wn_ref[...], preferred_element_type=jnp.float32)
        + b_ref[...]
    )
    if relu:
        h = jnp.maximum(h, 0.0)
    o_ref[...] = h


def _make_tc_combine(relu: bool, d_out: int):
    return pl.pallas_call(
        functools.partial(_combine_body, relu),
        grid=(N // R,),
        in_specs=[
            pl.BlockSpec((NC, R, D), lambda i: (0, i, 0)),
            pl.BlockSpec((NC, R, 1), lambda i: (0, i, 0)),
            pl.BlockSpec((R, D), lambda i: (i, 0)),
            pl.BlockSpec((D, d_out), lambda i: (0, 0)),
            pl.BlockSpec((D, d_out), lambda i: (0, 0)),
            pl.BlockSpec((1, d_out), lambda i: (0, 0)),
        ],
        out_shape=jax.ShapeDtypeStruct((N, d_out), jnp.float32),
        out_specs=pl.BlockSpec((R, d_out), lambda i: (i, 0)),
    )


_tc_combine_relu = _make_tc_combine(True, D)
_tc_final = _make_tc_combine(False, C_PAD)


def kernel(x, edge_index, W_self_0, W_neigh_0, b_0, W_self_1, W_neigh_1, b_1,
           W_self_2, W_neigh_2, b_2):
    pad = E_PAD - E
    src = jnp.concatenate([edge_index[0], jnp.arange(pad, dtype=jnp.int32) % N])
    dst = jnp.concatenate(
        [edge_index[1], N + (jnp.arange(pad, dtype=jnp.int32) % (N_PAD - N))]
    )

    ws2 = jnp.pad(W_self_2, ((0, 0), (0, C_PAD - C_OUT)))
    wn2 = jnp.pad(W_neigh_2, ((0, 0), (0, C_PAD - C_OUT)))
    b2 = jnp.pad(b_2, (0, C_PAD - C_OUT)).reshape(1, C_PAD)

    (degf,) = _sc_deg(dst)
    degp = degf.reshape(NC, N_PAD, D)[:, :, 0:1]

    (p0,) = _sc_agg(x, src, dst)
    p0 = p0.reshape(NC, N_PAD, D)
    h1 = _tc_combine_relu(p0, degp, x, W_self_0, W_neigh_0, b_0.reshape(1, D))

    (p1,) = _sc_agg(h1, src, dst)
    p1 = p1.reshape(NC, N_PAD, D)
    h2 = _tc_combine_relu(p1, degp, h1, W_self_1, W_neigh_1, b_1.reshape(1, D))

    (p2,) = _sc_agg(h2, src, dst)
    p2 = p2.reshape(NC, N_PAD, D)
    out = _tc_final(p2, degp, h2, ws2, wn2, b2)
    return out[:, :C_OUT]


# speedup vs baseline: 3.0768x; 1.4720x over previous
import functools

import jax
import jax.numpy as jnp
from jax import lax
from jax.experimental import pallas as pl
from jax.experimental.pallas import tpu as pltpu
from jax.experimental.pallas import tpu_sc as plsc

N = 10000
E = 320000
D = 128
C_OUT = 47
C_PAD = 48

NC = 2
NS = 16
NW = NC * NS
CH = 128
T = 80
E_PAD = NW * T * CH
N_PAD = 10240
ROWS_PER_TILE = N_PAD // NS
CP_PER_TILE = ROWS_PER_TILE // CH

_MESH = plsc.VectorSubcoreMesh(
    core_axis_name="c", subcore_axis_name="s", num_cores=NC, num_subcores=NS
)


def _zero_rows(rows):
    def zrow(i, carry):
        for j in range(D // 16):
            rows[i, pl.ds(j * 16, 16)] = jnp.zeros((16,), jnp.float32)
        return carry

    lax.fori_loop(0, CH, zrow, 0)


def _agg_body(tab_hbm, src_hbm, dst_hbm, out_hbm,
              is0, is1, id0, id1, rows0, rows1, acc, sg0, sg1, ss0, ss1):
    c = lax.axis_index("c")
    s = lax.axis_index("s")
    wid = c * NS + s
    idx_s = [is0, is1]
    idx_d = [id0, id1]
    rows = [rows0, rows1]
    sem_g = [sg0, sg1]
    sem_s = [ss0, ss1]

    _zero_rows(rows0)
    for k in range(CP_PER_TILE):
        pltpu.sync_copy(rows0, acc.at[pl.ds(s * ROWS_PER_TILE + k * CH, CH)])
    plsc.subcore_barrier()

    base = wid * (T * CH)

    def load_and_gather(b, off):
        pltpu.sync_copy(src_hbm.at[pl.ds(off, CH)], idx_s[b])
        pltpu.sync_copy(dst_hbm.at[pl.ds(off, CH)], idx_d[b])
        pltpu.async_copy(tab_hbm.at[idx_s[b]], rows[b], sem_g[b])

    load_and_gather(0, base)
    load_and_gather(1, base + CH)

    def edge_pair(k, carry):
        for b in range(2):
            i = 2 * k + b
            pltpu.make_async_copy(tab_hbm.at[idx_s[b]], rows[b], sem_g[b]).wait()
            pltpu.async_copy(rows[b], acc.at[idx_d[b]], sem_s[b], add=True)
            pltpu.make_async_copy(rows[b], acc.at[idx_d[b]], sem_s[b]).wait()

            @pl.when(k < T // 2 - 1)
            def _():
                load_and_gather(b, base + (i + 2) * CH)
        return carry

    lax.fori_loop(0, T // 2, edge_pair, 0)
    plsc.subcore_barrier()

    for k in range(CP_PER_TILE):
        r0 = s * ROWS_PER_TILE + k * CH
        pltpu.sync_copy(acc.at[pl.ds(r0, CH)], rows0)
        pltpu.sync_copy(rows0, out_hbm.at[pl.ds(c * N_PAD + r0, CH)])


def _deg_body(dst_hbm, out_hbm, idx_d, rows, acc, sem):
    c = lax.axis_index("c")
    s = lax.axis_index("s")
    wid = c * NS + s

    _zero_rows(rows)
    for k in range(CP_PER_TILE):
        pltpu.sync_copy(rows, acc.at[pl.ds(s * ROWS_PER_TILE + k * CH, CH)])
    plsc.subcore_barrier()

    def fill1(i, carry):
        for j in range(D // 16):
            rows[i, pl.ds(j * 16, 16)] = jnp.ones((16,), jnp.float32)
        return carry

    lax.fori_loop(0, CH, fill1, 0)

    base = wid * (T * CH)

    def edge_chunk(i, carry):
        off = base + i * CH
        pltpu.sync_copy(dst_hbm.at[pl.ds(off, CH)], idx_d)
        pltpu.sync_copy(rows, acc.at[idx_d], add=True)
        return carry

    lax.fori_loop(0, T, edge_chunk, 0)
    plsc.subcore_barrier()

    for k in range(CP_PER_TILE):
        r0 = s * ROWS_PER_TILE + k * CH
        pltpu.sync_copy(acc.at[pl.ds(r0, CH)], rows)
        pltpu.sync_copy(rows, out_hbm.at[pl.ds(c * N_PAD + r0, CH)])


_sc_agg = pl.kernel(
    _agg_body,
    out_type=[jax.ShapeDtypeStruct((NC * N_PAD, D), jnp.float32)],
    mesh=_MESH,
    scratch_types=(
        [pltpu.VMEM((CH,), jnp.int32)] * 4
        + [pltpu.VMEM((CH, D), jnp.float32)] * 2
        + [pltpu.VMEM_SHARED((N_PAD, D), jnp.float32)]
        + [pltpu.SemaphoreType.DMA] * 4
    ),
)

_sc_deg = pl.kernel(
    _deg_body,
    out_type=[jax.ShapeDtypeStruct((NC * N_PAD, D), jnp.float32)],
    mesh=_MESH,
    scratch_types=[
        pltpu.VMEM((CH,), jnp.int32),
        pltpu.VMEM((CH, D), jnp.float32),
        pltpu.VMEM_SHARED((N_PAD, D), jnp.float32),
        pltpu.SemaphoreType.DMA,
    ],
)

R = 1000


def _combine_body(relu, p_ref, d_ref, x_ref, ws_ref, wn_ref, b_ref, o_ref):
    deg = jnp.maximum(d_ref[0, :, 0] + d_ref[1, :, 0], 1.0)
    neigh = (p_ref[0] + p_ref[1]) / deg[:, None]
    h = (
        jnp.dot(x_ref[...], ws_ref[...], preferred_element_type=jnp.float32)
        + jnp.dot(neigh, wn_ref[...], preferred_element_type=jnp.float32)
        + b_ref[...]
    )
    if relu:
        h = jnp.maximum(h, 0.0)
    o_ref[...] = h


def _make_tc_combine(relu: bool, d_out: int):
    return pl.pallas_call(
        functools.partial(_combine_body, relu),
        grid=(N // R,),
        in_specs=[
            pl.BlockSpec((NC, R, D), lambda i: (0, i, 0)),
            pl.BlockSpec((NC, R, 1), lambda i: (0, i, 0)),
            pl.BlockSpec((R, D), lambda i: (i, 0)),
            pl.BlockSpec((D, d_out), lambda i: (0, 0)),
            pl.BlockSpec((D, d_out), lambda i: (0, 0)),
            pl.BlockSpec((1, d_out), lambda i: (0, 0)),
        ],
        out_shape=jax.ShapeDtypeStruct((N, d_out), jnp.float32),
        out_specs=pl.BlockSpec((R, d_out), lambda i: (i, 0)),
    )


_tc_combine_relu = _make_tc_combine(True, D)
_tc_final = _make_tc_combine(False, C_PAD)


def kernel(x, edge_index, W_self_0, W_neigh_0, b_0, W_self_1, W_neigh_1, b_1,
           W_self_2, W_neigh_2, b_2):
    pad = E_PAD - E
    src = jnp.concatenate([edge_index[0], jnp.arange(pad, dtype=jnp.int32) % N])
    dst = jnp.concatenate(
        [edge_index[1], N + (jnp.arange(pad, dtype=jnp.int32) % (N_PAD - N))]
    )

    ws2 = jnp.pad(W_self_2, ((0, 0), (0, C_PAD - C_OUT)))
    wn2 = jnp.pad(W_neigh_2, ((0, 0), (0, C_PAD - C_OUT)))
    b2 = jnp.pad(b_2, (0, C_PAD - C_OUT)).reshape(1, C_PAD)

    (degf,) = _sc_deg(dst)
    degp = degf.reshape(NC, N_PAD, D)[:, :, 0:1]

    (p0,) = _sc_agg(x, src, dst)
    p0 = p0.reshape(NC, N_PAD, D)
    h1 = _tc_combine_relu(p0, degp, x, W_self_0, W_neigh_0, b_0.reshape(1, D))

    (p1,) = _sc_agg(h1, src, dst)
    p1 = p1.reshape(NC, N_PAD, D)
    h2 = _tc_combine_relu(p1, degp, h1, W_self_1, W_neigh_1, b_1.reshape(1, D))

    (p2,) = _sc_agg(h2, src, dst)
    p2 = p2.reshape(NC, N_PAD, D)
    out = _tc_final(p2, degp, h2, ws2, wn2, b2)
    return out[:, :C_OUT]
